# baseline (speedup 1.0000x reference)
import functools
import jax
import jax.numpy as jnp
from jax import lax
from jax.experimental import pallas as pl
from jax.experimental.pallas import tpu as pltpu

NE_ = 8
NS_ = 4
H_ = 1024
SH_ = 256
N_ = 512
NG_ = NE_ * NS_
NA_ = 2 * N_
BR_ = 256
MAXB_ = 36
ND_ = MAXB_ * BR_


def _gelu(v):
    return 0.5 * v * (1.0 + lax.erf(v * 0.7071067811865476))


def _row(colv, n):
    i0 = lax.broadcasted_iota(jnp.int32, (n, n), 0).astype(jnp.float32)
    i1 = lax.broadcasted_iota(jnp.int32, (n, n), 1).astype(jnp.float32)
    eye = (i0 == i1).astype(jnp.float32)
    return lax.dot_general(colv, eye, (((0,), (0,)), ((), ())),
                           preferred_element_type=jnp.float32,
                           precision=lax.Precision.HIGHEST)


def _top1(p, k):
    m = jnp.max(p, axis=1, keepdims=True)
    ii = lax.broadcasted_iota(jnp.int32, p.shape, 1).astype(jnp.float32)
    a = jnp.min(jnp.where(p == m, ii, float(k)), axis=1, keepdims=True)
    return m, a


def _routing_kernel(x_ref, Wsr_ref, bsr_ref, Wr_ref, br_ref, Wsub_ref,
                    bsub_ref, seb_ref, eb_ref, coef_ref, dst_tok_ref,
                    dst_w_ref, gb_ref):
    f32 = jnp.float32
    x = x_ref[...]

    pr = jax.nn.sigmoid(
        jnp.dot(x, Wr_ref[...], preferred_element_type=f32)
        + br_ref[...] + eb_ref[...])
    psub = jax.nn.sigmoid(
        jnp.dot(x, Wsub_ref[...], preferred_element_type=f32)
        + bsub_ref[...] + seb_ref[...])
    psh = jax.nn.sigmoid(
        jnp.dot(x, Wsr_ref[...], preferred_element_type=f32)
        + bsr_ref[...] + seb_ref[...])

    m1, a1 = _top1(pr, NE_)
    iota_e = lax.broadcasted_iota(jnp.int32, (N_, NE_), 1).astype(f32)
    pr2 = jnp.where(iota_e == a1, -1.0, pr)
    m2, a2 = _top1(pr2, NE_)
    _, asub = _top1(psub, NS_)
    sw, ash = _top1(psh, NS_)

    iota_s = lax.broadcasted_iota(jnp.int32, (N_, NS_), 1).astype(f32)
    coef_ref[...] = jnp.where(iota_s == ash, sw, 0.0)

    g0 = a1 * NS_ + asub
    g1 = a2 * NS_ + asub
    gflat = jnp.concatenate([g0, g1], axis=0)
    iota_g = lax.broadcasted_iota(jnp.int32, (NA_, NG_), 1).astype(f32)
    G = (gflat == iota_g).astype(f32)

    ioa0 = lax.broadcasted_iota(jnp.int32, (NA_, NA_), 0)
    ioa1 = lax.broadcasted_iota(jnp.int32, (NA_, NA_), 1)
    Lm = (ioa0 >= ioa1).astype(f32)
    C = jnp.dot(Lm, G, preferred_element_type=f32,
                precision=lax.Precision.HIGHEST)
    rank = jnp.sum(G * C, axis=1, keepdims=True) - 1.0

    ones_a = jnp.ones((NA_, 1), f32)
    counts = lax.dot_general(G, ones_a, (((0,), (0,)), ((), ())),
                             preferred_element_type=f32,
                             precision=lax.Precision.HIGHEST)
    nblk = jnp.floor((counts + (BR_ - 1)) / BR_)
    iog0 = lax.broadcasted_iota(jnp.int32, (NG_, NG_), 0)
    iog1 = lax.broadcasted_iota(jnp.int32, (NG_, NG_), 1)
    Mstrict = (iog1 < iog0).astype(f32)
    bo = jnp.dot(Mstrict, nblk, preferred_element_type=f32,
                 precision=lax.Precision.HIGHEST)
    po = bo * BR_
    poa = jnp.dot(G, po, preferred_element_type=f32,
                  precision=lax.Precision.HIGHEST)
    pa = poa + rank

    pab = jnp.floor(pa / BR_)
    paj = pa - pab * BR_
    iota_bm = lax.broadcasted_iota(jnp.int32, (NA_, MAXB_), 1).astype(f32)
    iota_jm = lax.broadcasted_iota(jnp.int32, (NA_, BR_), 1).astype(f32)
    BM = (pab == iota_bm).astype(f32)
    JM = (paj == iota_jm).astype(f32)
    iota_a0 = lax.broadcasted_iota(jnp.int32, (NA_, MAXB_), 0)
    tok_mat = (iota_a0 % N_).astype(f32)
    w_col = jnp.concatenate([m1, m2], axis=0)
    dst_tok = lax.dot_general(BM * tok_mat, JM, (((0,), (0,)), ((), ())),
                              preferred_element_type=f32,
                              precision=lax.Precision.HIGHEST)
    dst_w = lax.dot_general(BM * w_col, JM, (((0,), (0,)), ((), ())),
                            preferred_element_type=f32,
                            precision=lax.Precision.HIGHEST)
    dst_tok_ref[...] = dst_tok.astype(jnp.int32)
    dst_w_ref[...] = dst_w

    iota_b = lax.broadcasted_iota(jnp.int32, (NG_, MAXB_), 1).astype(f32)
    gb_raw = jnp.sum((bo <= iota_b).astype(f32), axis=0, keepdims=True) - 1.0
    nbtot = jnp.sum(nblk)
    iog_m = lax.broadcasted_iota(jnp.int32, (NG_, MAXB_), 0).astype(f32)
    lastg = jnp.max(jnp.where(counts > 0, iog_m, -1.0))
    gb = jnp.where(iota_b[0:1] < nbtot, gb_raw, lastg)
    gb_ref[...] = jnp.concatenate(
        [gb, jnp.full((1, 1), nbtot, f32)], axis=1).astype(jnp.int32)


def _mlp_kernel(gb_ref, dst_tok_ref, x_ref, We1_ref, be1_ref, We2_ref,
                be2_ref, dst_w_ref, coef_ref, Ws1_ref, bs1_ref, Ws2_ref,
                bs2_ref, out_ref, acc_ref):
    f32 = jnp.float32
    b = pl.program_id(0)
    nbtot = gb_ref[MAXB_]

    @pl.when(b == 0)
    def _():
        x = x_ref[...]
        shared = jnp.zeros((N_, H_), f32)
        for s in range(NS_):
            hs = _gelu(jnp.dot(x, Ws1_ref[s], preferred_element_type=f32)
                       + bs1_ref[s])
            ys = (jnp.dot(hs, Ws2_ref[s], preferred_element_type=f32)
                  + bs2_ref[s])
            shared = shared + coef_ref[:, s:s + 1] * ys
        acc_ref[...] = shared

    @pl.when(b < nbtot)
    def _():
        tokrow = dst_tok_ref[0]
        io_n = lax.broadcasted_iota(jnp.int32, (N_, BR_), 0)
        ST = (io_n == tokrow).astype(f32)
        xg = lax.dot_general(ST, x_ref[...], (((0,), (0,)), ((), ())),
                             preferred_element_type=f32)
        h = _gelu(jnp.dot(xg, We1_ref[0, 0], preferred_element_type=f32)
                  + be1_ref[0, 0])
        y = (jnp.dot(h, We2_ref[0, 0], preferred_element_type=f32)
             + be2_ref[0, 0])
        io0 = lax.broadcasted_iota(jnp.int32, (BR_, BR_), 0)
        io1 = lax.broadcasted_iota(jnp.int32, (BR_, BR_), 1)
        diagw = jnp.where(io0 == io1, dst_w_ref[0], 0.0)
        wy = jnp.dot(diagw, y, preferred_element_type=f32)
        acc_ref[...] += jnp.dot(ST, wy, preferred_element_type=f32)

    @pl.when(b == MAXB_ - 1)
    def _():
        out_ref[...] = acc_ref[...]


def _routing_call(x, Wsr, bsr2, Wr, br2, Wsub, bsub2, seb2, eb2,
                  interpret=False):
    f32 = jnp.float32
    return pl.pallas_call(
        _routing_kernel,
        out_shape=(
            jax.ShapeDtypeStruct((N_, NS_), f32),
            jax.ShapeDtypeStruct((MAXB_, BR_), jnp.int32),
            jax.ShapeDtypeStruct((MAXB_, BR_), f32),
            jax.ShapeDtypeStruct((1, MAXB_ + 1), jnp.int32),
        ),
        interpret=interpret,
    )(x, Wsr, bsr2, Wr, br2, Wsub, bsub2, seb2, eb2)


def _mlp_call(gb, dst_tok3, x, We1, be1r, We2, be2r, dst_w3, coef, Ws1, bs1r,
              Ws2, bs2r, interpret=False):
    f32 = jnp.float32
    wspec = pl.BlockSpec((1, 1, H_, H_),
                         lambda b, gb: (gb[b] // NS_, gb[b] % NS_, 0, 0))
    bspec = pl.BlockSpec((1, 1, 1, H_),
                         lambda b, gb: (gb[b] // NS_, gb[b] % NS_, 0, 0))
    grid_spec = pltpu.PrefetchScalarGridSpec(
        num_scalar_prefetch=1,
        grid=(MAXB_,),
        in_specs=[
            pl.BlockSpec((1, 1, BR_), lambda b, gb: (b, 0, 0)),
            pl.BlockSpec((N_, H_), lambda b, gb: (0, 0)),
            wspec, bspec, wspec, bspec,
            pl.BlockSpec((1, 1, BR_), lambda b, gb: (b, 0, 0)),
            pl.BlockSpec((N_, NS_), lambda b, gb: (0, 0)),
            pl.BlockSpec((NS_, H_, SH_), lambda b, gb: (0, 0, 0)),
            pl.BlockSpec((NS_, 1, SH_), lambda b, gb: (0, 0, 0)),
            pl.BlockSpec((NS_, SH_, H_), lambda b, gb: (0, 0, 0)),
            pl.BlockSpec((NS_, 1, H_), lambda b, gb: (0, 0, 0)),
        ],
        out_specs=pl.BlockSpec((N_, H_), lambda b, gb: (0, 0)),
        scratch_shapes=[pltpu.VMEM((N_, H_), f32)],
    )
    return pl.pallas_call(
        _mlp_kernel,
        grid_spec=grid_spec,
        out_shape=jax.ShapeDtypeStruct((N_, H_), f32),
        interpret=interpret,
    )(gb, dst_tok3, x, We1, be1r, We2, be2r, dst_w3, coef, Ws1, bs1r, Ws2,
      bs2r)


@functools.partial(jax.jit, static_argnames=("interpret",))
def _kernel_impl(x, Ws1, bs1, Ws2, bs2, Wsr, bsr, We1, be1, We2, be2, Wr, br,
                 Wsub, bsub, expert_bias, sub_expert_bias, interpret=False):
    coef, dst_tok, dst_w, gb = _routing_call(
        x, Wsr, bsr.reshape(1, NS_), Wr, br.reshape(1, NE_), Wsub,
        bsub.reshape(1, NS_), sub_expert_bias.reshape(1, NS_),
        expert_bias.reshape(1, NE_), interpret=interpret)
    return _mlp_call(
        gb.reshape(MAXB_ + 1), dst_tok.reshape(MAXB_, 1, BR_), x, We1,
        be1.reshape(NE_, NS_, 1, H_), We2, be2.reshape(NE_, NS_, 1, H_),
        dst_w.reshape(MAXB_, 1, BR_), coef, Ws1, bs1.reshape(NS_, 1, SH_),
        Ws2, bs2.reshape(NS_, 1, H_), interpret=interpret)


def kernel(x, Ws1, bs1, Ws2, bs2, Wsr, bsr, We1, be1, We2, be2, Wr, br,
           Wsub, bsub, expert_bias, sub_expert_bias):
    return _kernel_impl(x, Ws1, bs1, Ws2, bs2, Wsr, bsr, We1, be1, We2, be2,
                        Wr, br, Wsub, bsub, expert_bias, sub_expert_bias)


# speedup vs baseline: 1.1232x; 1.1232x over previous
import functools
import jax
import jax.numpy as jnp
from jax import lax
from jax.experimental import pallas as pl
from jax.experimental.pallas import tpu as pltpu

NE_ = 8
NS_ = 4
H_ = 1024
SH_ = 256
N_ = 512
NG_ = NE_ * NS_
NA_ = 2 * N_
BR_ = 128
MAXB_ = 40
ND_ = MAXB_ * BR_


def _gelu(v):
    return 0.5 * v * (1.0 + lax.erf(v * 0.7071067811865476))


def _row(colv, n):
    i0 = lax.broadcasted_iota(jnp.int32, (n, n), 0).astype(jnp.float32)
    i1 = lax.broadcasted_iota(jnp.int32, (n, n), 1).astype(jnp.float32)
    eye = (i0 == i1).astype(jnp.float32)
    return lax.dot_general(colv, eye, (((0,), (0,)), ((), ())),
                           preferred_element_type=jnp.float32,
                           precision=lax.Precision.HIGHEST)


def _top1(p, k):
    m = jnp.max(p, axis=1, keepdims=True)
    ii = lax.broadcasted_iota(jnp.int32, p.shape, 1).astype(jnp.float32)
    a = jnp.min(jnp.where(p == m, ii, float(k)), axis=1, keepdims=True)
    return m, a


def _routing_kernel(x_ref, Wsr_ref, bsr_ref, Wr_ref, br_ref, Wsub_ref,
                    bsub_ref, seb_ref, eb_ref, coef_ref, dst_tok_ref,
                    dst_w_ref, gb_ref):
    f32 = jnp.float32
    x = x_ref[...]

    pr = jax.nn.sigmoid(
        jnp.dot(x, Wr_ref[...], preferred_element_type=f32)
        + br_ref[...] + eb_ref[...])
    psub = jax.nn.sigmoid(
        jnp.dot(x, Wsub_ref[...], preferred_element_type=f32)
        + bsub_ref[...] + seb_ref[...])
    psh = jax.nn.sigmoid(
        jnp.dot(x, Wsr_ref[...], preferred_element_type=f32)
        + bsr_ref[...] + seb_ref[...])

    m1, a1 = _top1(pr, NE_)
    iota_e = lax.broadcasted_iota(jnp.int32, (N_, NE_), 1).astype(f32)
    pr2 = jnp.where(iota_e == a1, -1.0, pr)
    m2, a2 = _top1(pr2, NE_)
    _, asub = _top1(psub, NS_)
    sw, ash = _top1(psh, NS_)

    iota_s = lax.broadcasted_iota(jnp.int32, (N_, NS_), 1).astype(f32)
    coef_ref[...] = jnp.where(iota_s == ash, sw, 0.0)

    g0 = a1 * NS_ + asub
    g1 = a2 * NS_ + asub
    gflat = jnp.concatenate([g0, g1], axis=0)
    iota_g = lax.broadcasted_iota(jnp.int32, (NA_, NG_), 1).astype(f32)
    G = (gflat == iota_g).astype(f32)

    ioa0 = lax.broadcasted_iota(jnp.int32, (NA_, NA_), 0)
    ioa1 = lax.broadcasted_iota(jnp.int32, (NA_, NA_), 1)
    Lm = (ioa0 >= ioa1).astype(f32)
    C = jnp.dot(Lm, G, preferred_element_type=f32,
                precision=lax.Precision.HIGHEST)
    rank = jnp.sum(G * C, axis=1, keepdims=True) - 1.0

    ones_a = jnp.ones((NA_, 1), f32)
    counts = lax.dot_general(G, ones_a, (((0,), (0,)), ((), ())),
                             preferred_element_type=f32,
                             precision=lax.Precision.HIGHEST)
    nblk = jnp.floor((counts + (BR_ - 1)) / BR_)
    iog0 = lax.broadcasted_iota(jnp.int32, (NG_, NG_), 0)
    iog1 = lax.broadcasted_iota(jnp.int32, (NG_, NG_), 1)
    Mstrict = (iog1 < iog0).astype(f32)
    bo = jnp.dot(Mstrict, nblk, preferred_element_type=f32,
                 precision=lax.Precision.HIGHEST)
    po = bo * BR_
    poa = jnp.dot(G, po, preferred_element_type=f32,
                  precision=lax.Precision.HIGHEST)
    pa = poa + rank

    pab = jnp.floor(pa / BR_)
    paj = pa - pab * BR_
    iota_bm = lax.broadcasted_iota(jnp.int32, (NA_, MAXB_), 1).astype(f32)
    iota_jm = lax.broadcasted_iota(jnp.int32, (NA_, BR_), 1).astype(f32)
    BM = (pab == iota_bm).astype(f32)
    JM = (paj == iota_jm).astype(f32)
    iota_a0 = lax.broadcasted_iota(jnp.int32, (NA_, MAXB_), 0)
    tok_mat = (iota_a0 % N_).astype(f32)
    w_col = jnp.concatenate([m1, m2], axis=0)
    dst_tok = lax.dot_general(BM * tok_mat, JM, (((0,), (0,)), ((), ())),
                              preferred_element_type=f32,
                              precision=lax.Precision.HIGHEST)
    dst_w = lax.dot_general(BM * w_col, JM, (((0,), (0,)), ((), ())),
                            preferred_element_type=f32,
                            precision=lax.Precision.HIGHEST)
    dst_tok_ref[...] = dst_tok.astype(jnp.int32)
    dst_w_ref[...] = dst_w

    iota_b = lax.broadcasted_iota(jnp.int32, (NG_, MAXB_), 1).astype(f32)
    gb_raw = jnp.sum((bo <= iota_b).astype(f32), axis=0, keepdims=True) - 1.0
    nbtot = jnp.sum(nblk)
    iog_m = lax.broadcasted_iota(jnp.int32, (NG_, MAXB_), 0).astype(f32)
    lastg = jnp.max(jnp.where(counts > 0, iog_m, -1.0))
    gb = jnp.where(iota_b[0:1] < nbtot, gb_raw, lastg)
    gb_ref[...] = jnp.concatenate(
        [gb, jnp.full((1, 1), nbtot, f32)], axis=1).astype(jnp.int32)


def _mlp_kernel(gb_ref, dst_tok_ref, x_ref, We1_ref, be1_ref, We2_ref,
                be2_ref, dst_w_ref, coef_ref, Ws1_ref, bs1_ref, Ws2_ref,
                bs2_ref, out_ref, acc_ref):
    f32 = jnp.float32
    b = pl.program_id(0)
    nbtot = gb_ref[MAXB_]

    @pl.when(b == 0)
    def _():
        x = x_ref[...]
        shared = jnp.zeros((N_, H_), f32)
        for s in range(NS_):
            hs = _gelu(jnp.dot(x, Ws1_ref[s], preferred_element_type=f32)
                       + bs1_ref[s])
            ys = (jnp.dot(hs, Ws2_ref[s], preferred_element_type=f32)
                  + bs2_ref[s])
            shared = shared + coef_ref[:, s:s + 1] * ys
        acc_ref[...] = shared

    @pl.when(b < nbtot)
    def _():
        tokrow = dst_tok_ref[0]
        io_n = lax.broadcasted_iota(jnp.int32, (N_, BR_), 0)
        ST = (io_n == tokrow).astype(f32)
        xg = lax.dot_general(ST, x_ref[...], (((0,), (0,)), ((), ())),
                             preferred_element_type=f32)
        h = _gelu(jnp.dot(xg, We1_ref[0, 0], preferred_element_type=f32)
                  + be1_ref[0, 0])
        y = (jnp.dot(h, We2_ref[0, 0], preferred_element_type=f32)
             + be2_ref[0, 0])
        io0 = lax.broadcasted_iota(jnp.int32, (BR_, BR_), 0)
        io1 = lax.broadcasted_iota(jnp.int32, (BR_, BR_), 1)
        diagw = jnp.where(io0 == io1, dst_w_ref[0], 0.0)
        wy = jnp.dot(diagw, y, preferred_element_type=f32)
        acc_ref[...] += jnp.dot(ST, wy, preferred_element_type=f32)

    @pl.when(b == MAXB_ - 1)
    def _():
        out_ref[...] = acc_ref[...]


def _routing_call(x, Wsr, bsr2, Wr, br2, Wsub, bsub2, seb2, eb2,
                  interpret=False):
    f32 = jnp.float32
    return pl.pallas_call(
        _routing_kernel,
        out_shape=(
            jax.ShapeDtypeStruct((N_, NS_), f32),
            jax.ShapeDtypeStruct((MAXB_, BR_), jnp.int32),
            jax.ShapeDtypeStruct((MAXB_, BR_), f32),
            jax.ShapeDtypeStruct((1, MAXB_ + 1), jnp.int32),
        ),
        interpret=interpret,
    )(x, Wsr, bsr2, Wr, br2, Wsub, bsub2, seb2, eb2)


def _mlp_call(gb, dst_tok3, x, We1, be1r, We2, be2r, dst_w3, coef, Ws1, bs1r,
              Ws2, bs2r, interpret=False):
    f32 = jnp.float32
    wspec = pl.BlockSpec((1, 1, H_, H_),
                         lambda b, gb: (gb[b] // NS_, gb[b] % NS_, 0, 0))
    bspec = pl.BlockSpec((1, 1, 1, H_),
                         lambda b, gb: (gb[b] // NS_, gb[b] % NS_, 0, 0))
    grid_spec = pltpu.PrefetchScalarGridSpec(
        num_scalar_prefetch=1,
        grid=(MAXB_,),
        in_specs=[
            pl.BlockSpec((1, 1, BR_), lambda b, gb: (b, 0, 0)),
            pl.BlockSpec((N_, H_), lambda b, gb: (0, 0)),
            wspec, bspec, wspec, bspec,
            pl.BlockSpec((1, 1, BR_), lambda b, gb: (b, 0, 0)),
            pl.BlockSpec((N_, NS_), lambda b, gb: (0, 0)),
            pl.BlockSpec((NS_, H_, SH_), lambda b, gb: (0, 0, 0)),
            pl.BlockSpec((NS_, 1, SH_), lambda b, gb: (0, 0, 0)),
            pl.BlockSpec((NS_, SH_, H_), lambda b, gb: (0, 0, 0)),
            pl.BlockSpec((NS_, 1, H_), lambda b, gb: (0, 0, 0)),
        ],
        out_specs=pl.BlockSpec((N_, H_), lambda b, gb: (0, 0)),
        scratch_shapes=[pltpu.VMEM((N_, H_), f32)],
    )
    return pl.pallas_call(
        _mlp_kernel,
        grid_spec=grid_spec,
        out_shape=jax.ShapeDtypeStruct((N_, H_), f32),
        interpret=interpret,
    )(gb, dst_tok3, x, We1, be1r, We2, be2r, dst_w3, coef, Ws1, bs1r, Ws2,
      bs2r)


@functools.partial(jax.jit, static_argnames=("interpret",))
def _kernel_impl(x, Ws1, bs1, Ws2, bs2, Wsr, bsr, We1, be1, We2, be2, Wr, br,
                 Wsub, bsub, expert_bias, sub_expert_bias, interpret=False):
    coef, dst_tok, dst_w, gb = _routing_call(
        x, Wsr, bsr.reshape(1, NS_), Wr, br.reshape(1, NE_), Wsub,
        bsub.reshape(1, NS_), sub_expert_bias.reshape(1, NS_),
        expert_bias.reshape(1, NE_), interpret=interpret)
    return _mlp_call(
        gb.reshape(MAXB_ + 1), dst_tok.reshape(MAXB_, 1, BR_), x, We1,
        be1.reshape(NE_, NS_, 1, H_), We2, be2.reshape(NE_, NS_, 1, H_),
        dst_w.reshape(MAXB_, 1, BR_), coef, Ws1, bs1.reshape(NS_, 1, SH_),
        Ws2, bs2.reshape(NS_, 1, H_), interpret=interpret)


def kernel(x, Ws1, bs1, Ws2, bs2, Wsr, bsr, We1, be1, We2, be2, Wr, br,
           Wsub, bsub, expert_bias, sub_expert_bias):
    return _kernel_impl(x, Ws1, bs1, Ws2, bs2, Wsr, bsr, We1, be1, We2, be2,
                        Wr, br, Wsub, bsub, expert_bias, sub_expert_bias)
